# baseline (speedup 1.0000x reference)
import functools

import jax
import jax.numpy as jnp
from jax import lax
from jax.experimental import pallas as pl
from jax.experimental.pallas import tpu as pltpu
from jax.experimental.pallas import tpu_sc as plsc

VOCAB = 100000
B = 16384
D = 128
H = 256

NC = 2
NS = 16
NW = NC * NS
CHUNK = 128
CH_PER_TABLE = (B // NW) // CHUNK
NBUF = 3


def _sc_gather(uids2d, vids2d, user_table, video_table):
    mesh = plsc.VectorSubcoreMesh(core_axis_name="c", subcore_axis_name="s")

    @functools.partial(
        pl.kernel,
        out_type=jax.ShapeDtypeStruct((2 * B, D), jnp.float32),
        mesh=mesh,
        scratch_types=[
            pltpu.VMEM((2 * CH_PER_TABLE, CHUNK), jnp.int32),
            [pltpu.VMEM((2 * CHUNK, D), jnp.float32) for _ in range(NBUF)],
            [pltpu.SemaphoreType.DMA for _ in range(NBUF)],
            [pltpu.SemaphoreType.DMA for _ in range(NBUF)],
        ],
    )
    def k(uids_hbm, vids_hbm, utab_hbm, vtab_hbm, out_hbm,
          idx_v, bufs, sems, wsems):
        wid = lax.axis_index("s") * NC + lax.axis_index("c")
        row0 = wid * CH_PER_TABLE
        icp_u = pltpu.make_async_copy(uids_hbm.at[pl.ds(row0, CH_PER_TABLE)],
                                      idx_v.at[pl.ds(0, CH_PER_TABLE)], wsems[0])
        icp_v = pltpu.make_async_copy(vids_hbm.at[pl.ds(row0, CH_PER_TABLE)],
                                      idx_v.at[pl.ds(CH_PER_TABLE, CH_PER_TABLE)],
                                      wsems[1])
        icp_u.start()
        icp_v.start()
        icp_u.wait()
        icp_v.wait()

        tabs = (utab_hbm, vtab_hbm)
        half = CH_PER_TABLE // 2
        total = 4 * half
        gathers = [None] * total
        scatters = [None] * (2 * half)

        def scatter_prev(dj):
            gathers[2 * dj].wait()
            gathers[2 * dj + 1].wait()
            t = dj // half
            base = t * B + wid * (B // NW) + (dj % half) * 2 * CHUNK
            sc = pltpu.make_async_copy(
                bufs[dj % NBUF], out_hbm.at[pl.ds(base, 2 * CHUNK)],
                wsems[dj % NBUF])
            sc.start()
            scatters[dj] = sc

        for dj in range(2 * half):
            b = dj % NBUF
            if dj >= NBUF:
                scatters[dj - NBUF].wait()
            t = dj // half
            for hh in range(2):
                j = 2 * dj + hh
                cp = pltpu.make_async_copy(
                    tabs[t].at[idx_v.at[t * CH_PER_TABLE + (dj % half) * 2 + hh]],
                    bufs[b].at[pl.ds(hh * CHUNK, CHUNK)], sems[b])
                cp.start()
                gathers[j] = cp
            if dj > 0:
                scatter_prev(dj - 1)
        scatter_prev(2 * half - 1)
        for dj in range(2 * half - NBUF, 2 * half):
            scatters[dj].wait()

    return k(uids2d, vids2d, user_table, video_table)


BLK = 4096


def _towers_body(xu_ref, xv_ref,
                 uw1_ref, ub1_ref, uw2_ref, ub2_ref,
                 vw1_ref, vb1_ref, vw2_ref, vb2_ref, out_ref):
    for t, (x_ref, w1_ref, b1_ref, w2_ref, b2_ref) in enumerate((
            (xu_ref, uw1_ref, ub1_ref, uw2_ref, ub2_ref),
            (xv_ref, vw1_ref, vb1_ref, vw2_ref, vb2_ref))):
        x = x_ref[...].astype(jnp.bfloat16)
        h = jnp.dot(x, w1_ref[...].astype(jnp.bfloat16),
                    preferred_element_type=jnp.float32)
        h = jnp.maximum(h + b1_ref[...], 0.0).astype(jnp.bfloat16)
        y = jnp.dot(h, w2_ref[...].astype(jnp.bfloat16),
                    preferred_element_type=jnp.float32)
        y = y + b2_ref[...]
        ss = jnp.sum(y * y, axis=1, keepdims=True)
        out_ref[t] = y * lax.rsqrt(jnp.maximum(ss, 1e-12))


def _tc_towers(emb, uW1, ub1, uW2, ub2, vW1, vb1, vW2, vb2):
    nblk = B // BLK
    wspec1 = pl.BlockSpec((D, H), lambda i: (0, 0))
    bspec1 = pl.BlockSpec((1, H), lambda i: (0, 0))
    wspec2 = pl.BlockSpec((H, D), lambda i: (0, 0))
    bspec2 = pl.BlockSpec((1, D), lambda i: (0, 0))
    return pl.pallas_call(
        _towers_body,
        grid=(nblk,),
        in_specs=[
            pl.BlockSpec((BLK, D), lambda i: (i, 0)),
            pl.BlockSpec((BLK, D), lambda i, n=nblk: (n + i, 0)),
            wspec1, bspec1, wspec2, bspec2,
            wspec1, bspec1, wspec2, bspec2,
        ],
        out_specs=pl.BlockSpec((2, BLK, D), lambda i: (0, i, 0)),
        out_shape=jax.ShapeDtypeStruct((2, B, D), jnp.float32),
    )(emb, emb, uW1, ub1, uW2, ub2, vW1, vb1, vW2, vb2)


def kernel(user_ids, video_ids, user_table, video_table,
           uW1, ub1, uW2, ub2, vW1, vb1, vW2, vb2):
    uids2d = user_ids.astype(jnp.int32).reshape(B // CHUNK, CHUNK)
    vids2d = video_ids.astype(jnp.int32).reshape(B // CHUNK, CHUNK)
    emb = _sc_gather(uids2d, vids2d, user_table, video_table)
    return _tc_towers(
        emb,
        uW1, ub1[None, :], uW2, ub2[None, :],
        vW1, vb1[None, :], vW2, vb2[None, :],
    )
